# baseline (speedup 1.0000x reference)
import functools

import jax
import jax.numpy as jnp
from jax import lax
from jax.experimental import pallas as pl
from jax.experimental.pallas import tpu as pltpu
from jax.experimental.pallas import tpu_sc as plsc

D, H, W = 96, 512, 512
P = H * W
K = 1024
NC, NS, L = 2, 16, 16
NT = NC * NS
CPT = D // NT
SR = 8
CHUNK = SR * W
NCHUNK = H // SR
VPR = W // L
KROWS = K // L

_mesh = plsc.VectorSubcoreMesh(core_axis_name="c", subcore_axis_name="s")


@functools.partial(
    pl.kernel,
    out_type=jax.ShapeDtypeStruct((NT, L), jnp.float32),
    mesh=_mesh,
    compiler_params=pltpu.CompilerParams(needs_layout_passes=False,
                                         use_tc_tiling_on_sc=True),
    scratch_types=[
        pltpu.VMEM((2 * SR, W), jnp.int32),
        pltpu.VMEM((2 * CPT * SR, W), jnp.float32),
        pltpu.VMEM((CPT * K,), jnp.float32),
        pltpu.VMEM((K,), jnp.float32),
        pltpu.VMEM((NS, K), jnp.float32),
        pltpu.VMEM((L,), jnp.float32),
        pltpu.VMEM_SHARED((NS, K), jnp.float32),
        pltpu.SemaphoreType.DMA,
        pltpu.SemaphoreType.DMA,
    ],
)
def _penalty_kernel(emb, seg, out, seg2, x2, bucket, counts1d, cbuf,
                    res_v, shared_cnt, semA, semB):
    cid = lax.axis_index("c")
    sid = lax.axis_index("s")
    wid = cid * NS + sid

    zeros = jnp.zeros((L,), jnp.float32)
    ones = jnp.ones((L,), jnp.float32)

    for r in range(KROWS):
        counts1d[pl.ds(r * L, L)] = zeros
    for r in range(CPT * K // L):
        bucket[pl.ds(r * L, L)] = zeros

    d0 = wid * CPT
    sems = (semA, semB)

    def issue(b, j):
        sem = sems[b]
        pltpu.async_copy(seg.at[pl.ds(j * SR, SR), :],
                         seg2.at[pl.ds(b * SR, SR), :], sem)
        for dd in range(CPT):
            pltpu.async_copy(emb.at[d0 + dd, pl.ds(j * SR, SR), :],
                             x2.at[pl.ds((b * CPT + dd) * SR, SR), :], sem)

    def drain(b, j):
        sem = sems[b]
        pltpu.make_async_copy(seg.at[pl.ds(j * SR, SR), :],
                              seg2.at[pl.ds(b * SR, SR), :], sem).wait()
        for dd in range(CPT):
            pltpu.make_async_copy(emb.at[d0 + dd, pl.ds(j * SR, SR), :],
                                  x2.at[pl.ds((b * CPT + dd) * SR, SR), :],
                                  sem).wait()

    UNR = 3

    def process(b, j, sq):
        @pl.when(lax.rem(j, NS) == sid)
        def _():
            @plsc.parallel_loop(0, VPR, unroll=UNR * CPT)
            def _(i):
                for r in range(SR):
                    idx = seg2[b * SR + r, pl.ds(i * L, L)]
                    plsc.addupdate_scatter(counts1d, [idx], ones)

        @plsc.parallel_loop(0, VPR, unroll=UNR, carry=tuple(sq))
        def accs(i, accs):
            accs = list(accs)
            for r in range(SR):
                idx = seg2[b * SR + r, pl.ds(i * L, L)]
                for dd in range(CPT):
                    x = x2[(b * CPT + dd) * SR + r, pl.ds(i * L, L)]
                    plsc.addupdate_scatter(
                        bucket.at[pl.ds(dd * K, K)], [idx], x)
                    accs[dd] = accs[dd] + x * x
            return tuple(accs)

        return accs

    issue(0, jnp.int32(0))
    issue(1, jnp.int32(1))

    def pair(t, sq):
        for b in range(2):
            j = 2 * t + b
            drain(b, j)
            sq = process(b, j, sq)

            @pl.when(j + 2 < NCHUNK)
            def _():
                issue(b, j + 2)
        return sq

    accs = lax.fori_loop(0, NCHUNK // 2, pair,
                         tuple(zeros for _ in range(CPT)))
    sq = accs[0]
    for a in accs[1:]:
        sq = sq + a

    pltpu.sync_copy(counts1d, shared_cnt.at[sid])
    plsc.subcore_barrier()
    pltpu.sync_copy(shared_cnt, cbuf)

    def sum_row(r, _):
        acc = zeros
        for s in range(NS):
            acc = acc + cbuf[s, pl.ds(r * L, L)]
        counts1d[pl.ds(r * L, L)] = acc
        return 0

    lax.fori_loop(0, KROWS, sum_row, 0)

    def term_row(r, acc):
        c = counts1d[pl.ds(r * L, L)]
        inv = 1.0 / jnp.maximum(c, 1.0)
        for dd in range(CPT):
            s = bucket[pl.ds(dd * K + r * L, L)]
            acc = acc + s * s * inv
        return acc

    acc = lax.fori_loop(0, KROWS, term_row, zeros)

    res_v[...] = sq - acc
    pltpu.sync_copy(res_v, out.at[wid])


def kernel(embedding, superpixel):
    parts = _penalty_kernel(embedding, superpixel.astype(jnp.int32))
    return jnp.sum(parts)


# speedup vs baseline: 1.3564x; 1.3564x over previous
import functools

import jax
import jax.numpy as jnp
from jax import lax
from jax.experimental import pallas as pl
from jax.experimental.pallas import tpu as pltpu
from jax.experimental.pallas import tpu_sc as plsc

D, H, W = 96, 512, 512
P = H * W
K = 1024
NC, NS, L = 2, 16, 16
NT = NC * NS
CPT = D // NT
SR = 8
CHUNK = SR * W
NCHUNK = H // SR
VPR = W // L
KROWS = K // L

_mesh = plsc.VectorSubcoreMesh(core_axis_name="c", subcore_axis_name="s")


@functools.partial(
    pl.kernel,
    out_type=jax.ShapeDtypeStruct((NT, L), jnp.float32),
    mesh=_mesh,
    compiler_params=pltpu.CompilerParams(needs_layout_passes=False,
                                         use_tc_tiling_on_sc=True),
    scratch_types=[
        pltpu.VMEM((2 * SR, W), jnp.int32),
        pltpu.VMEM((2 * CPT * SR, W), jnp.float32),
        pltpu.VMEM((CPT * K,), jnp.float32),
        pltpu.VMEM((K,), jnp.float32),
        pltpu.VMEM((NS, K), jnp.float32),
        pltpu.VMEM((L,), jnp.float32),
        pltpu.VMEM_SHARED((NS, K), jnp.float32),
        pltpu.SemaphoreType.DMA,
        pltpu.SemaphoreType.DMA,
    ],
)
def _penalty_kernel(emb, seg, out, seg2, x2, bucket, counts1d, cbuf,
                    res_v, shared_cnt, semA, semB):
    cid = lax.axis_index("c")
    sid = lax.axis_index("s")
    wid = cid * NS + sid

    zeros = jnp.zeros((L,), jnp.float32)
    ones = jnp.ones((L,), jnp.float32)

    for r in range(KROWS):
        counts1d[pl.ds(r * L, L)] = zeros
    for r in range(CPT * K // L):
        bucket[pl.ds(r * L, L)] = zeros

    d0 = wid * CPT
    sems = (semA, semB)

    def issue(b, j):
        sem = sems[b]
        pltpu.async_copy(seg.at[pl.ds(j * SR, SR), :],
                         seg2.at[pl.ds(b * SR, SR), :], sem)
        for dd in range(CPT):
            pltpu.async_copy(emb.at[d0 + dd, pl.ds(j * SR, SR), :],
                             x2.at[pl.ds((b * CPT + dd) * SR, SR), :], sem)

    def drain(b, j):
        sem = sems[b]
        pltpu.make_async_copy(seg.at[pl.ds(j * SR, SR), :],
                              seg2.at[pl.ds(b * SR, SR), :], sem).wait()
        for dd in range(CPT):
            pltpu.make_async_copy(emb.at[d0 + dd, pl.ds(j * SR, SR), :],
                                  x2.at[pl.ds((b * CPT + dd) * SR, SR), :],
                                  sem).wait()

    UNR = 1

    def process(b, j, sq):
        @pl.when(lax.rem(j, NS) == sid)
        def _():
            @plsc.parallel_loop(0, VPR, unroll=4)
            def _(i):
                for r in range(SR):
                    idx = seg2[b * SR + r, pl.ds(i * L, L)]
                    plsc.addupdate_scatter(counts1d, [idx], ones)

        @plsc.parallel_loop(0, VPR, unroll=UNR, carry=tuple(sq))
        def accs(i, accs):
            accs = list(accs)
            for r in range(SR):
                idx = seg2[b * SR + r, pl.ds(i * L, L)]
                for dd in range(CPT):
                    x = x2[(b * CPT + dd) * SR + r, pl.ds(i * L, L)]
                    plsc.addupdate_scatter(
                        bucket.at[pl.ds(dd * K, K)], [idx], x)
                    accs[dd] = accs[dd] + x * x
            return tuple(accs)

        return accs

    issue(0, jnp.int32(0))
    issue(1, jnp.int32(1))

    def pair(t, sq):
        for b in range(2):
            j = 2 * t + b
            drain(b, j)
            sq = process(b, j, sq)

            @pl.when(j + 2 < NCHUNK)
            def _():
                issue(b, j + 2)
        return sq

    accs = lax.fori_loop(0, NCHUNK // 2, pair,
                         tuple(zeros for _ in range(CPT)))
    sq = accs[0]
    for a in accs[1:]:
        sq = sq + a

    pltpu.sync_copy(counts1d, shared_cnt.at[sid])
    plsc.subcore_barrier()
    pltpu.sync_copy(shared_cnt, cbuf)

    def sum_row(r, _):
        acc = zeros
        for s in range(NS):
            acc = acc + cbuf[s, pl.ds(r * L, L)]
        counts1d[pl.ds(r * L, L)] = acc
        return 0

    lax.fori_loop(0, KROWS, sum_row, 0)

    def term_row(r, acc):
        c = counts1d[pl.ds(r * L, L)]
        inv = 1.0 / jnp.maximum(c, 1.0)
        for dd in range(CPT):
            s = bucket[pl.ds(dd * K + r * L, L)]
            acc = acc + s * s * inv
        return acc

    acc = lax.fori_loop(0, KROWS, term_row, zeros)

    res_v[...] = sq - acc
    pltpu.sync_copy(res_v, out.at[wid])


def kernel(embedding, superpixel):
    parts = _penalty_kernel(embedding, superpixel.astype(jnp.int32))
    return jnp.sum(parts)


# speedup vs baseline: 1.3567x; 1.0002x over previous
import functools

import jax
import jax.numpy as jnp
from jax import lax
from jax.experimental import pallas as pl
from jax.experimental.pallas import tpu as pltpu
from jax.experimental.pallas import tpu_sc as plsc

D, H, W = 96, 512, 512
P = H * W
K = 1024
NC, NS, L = 2, 16, 16
NT = NC * NS
CPT = D // NT
SR = 8
CHUNK = SR * W
NCHUNK = H // SR
VPR = W // L
KROWS = K // L

_mesh = plsc.VectorSubcoreMesh(core_axis_name="c", subcore_axis_name="s")


@functools.partial(
    pl.kernel,
    out_type=jax.ShapeDtypeStruct((NT, L), jnp.float32),
    mesh=_mesh,
    compiler_params=pltpu.CompilerParams(needs_layout_passes=False,
                                         use_tc_tiling_on_sc=True),
    scratch_types=[
        pltpu.VMEM((2 * SR, W), jnp.int32),
        pltpu.VMEM((2 * CPT * SR, W), jnp.float32),
        pltpu.VMEM((CPT * K,), jnp.float32),
        pltpu.VMEM((K,), jnp.float32),
        pltpu.VMEM((NS, K), jnp.float32),
        pltpu.VMEM((L,), jnp.float32),
        pltpu.VMEM_SHARED((NS, K), jnp.float32),
        pltpu.SemaphoreType.DMA,
        pltpu.SemaphoreType.DMA,
    ],
)
def _penalty_kernel(emb, seg, out, seg2, x2, bucket, counts1d, cbuf,
                    res_v, shared_cnt, semA, semB):
    cid = lax.axis_index("c")
    sid = lax.axis_index("s")
    wid = cid * NS + sid

    zeros = jnp.zeros((L,), jnp.float32)
    ones = jnp.ones((L,), jnp.float32)

    for r in range(KROWS):
        counts1d[pl.ds(r * L, L)] = zeros
    for r in range(CPT * K // L):
        bucket[pl.ds(r * L, L)] = zeros

    d0 = wid * CPT
    sems = (semA, semB)

    def issue(b, j):
        sem = sems[b]
        pltpu.async_copy(seg.at[pl.ds(j * SR, SR), :],
                         seg2.at[pl.ds(b * SR, SR), :], sem)
        for dd in range(CPT):
            pltpu.async_copy(emb.at[d0 + dd, pl.ds(j * SR, SR), :],
                             x2.at[pl.ds((b * CPT + dd) * SR, SR), :], sem)

    def drain(b, j):
        sem = sems[b]
        pltpu.make_async_copy(seg.at[pl.ds(j * SR, SR), :],
                              seg2.at[pl.ds(b * SR, SR), :], sem).wait()
        for dd in range(CPT):
            pltpu.make_async_copy(emb.at[d0 + dd, pl.ds(j * SR, SR), :],
                                  x2.at[pl.ds((b * CPT + dd) * SR, SR), :],
                                  sem).wait()

    UNR = 1

    def process(b, j, sq):
        @pl.when(lax.rem(j, NS) == sid)
        def _():
            @plsc.parallel_loop(0, VPR, unroll=4)
            def _(i):
                for r in range(SR):
                    idx = seg2[b * SR + r, pl.ds(i * L, L)]
                    plsc.addupdate_scatter(counts1d, [idx], ones)

        for rh in range(2):
            @plsc.parallel_loop(0, VPR, unroll=UNR, carry=tuple(sq))
            def accs(i, accs, rh=rh):
                accs = list(accs)
                for r in range(rh * SR // 2, (rh + 1) * SR // 2):
                    idx = seg2[b * SR + r, pl.ds(i * L, L)]
                    for dd in range(CPT):
                        x = x2[(b * CPT + dd) * SR + r, pl.ds(i * L, L)]
                        plsc.addupdate_scatter(
                            bucket.at[pl.ds(dd * K, K)], [idx], x)
                        accs[dd] = accs[dd] + x * x
                return tuple(accs)

            sq = accs

        return sq

    issue(0, jnp.int32(0))
    issue(1, jnp.int32(1))

    def pair(t, sq):
        for b in range(2):
            j = 2 * t + b
            drain(b, j)
            sq = process(b, j, sq)

            @pl.when(j + 2 < NCHUNK)
            def _():
                issue(b, j + 2)
        return sq

    accs = lax.fori_loop(0, NCHUNK // 2, pair,
                         tuple(zeros for _ in range(CPT)))
    sq = accs[0]
    for a in accs[1:]:
        sq = sq + a

    pltpu.sync_copy(counts1d, shared_cnt.at[sid])
    plsc.subcore_barrier()
    pltpu.sync_copy(shared_cnt, cbuf)

    def sum_row(r, _):
        acc = zeros
        for s in range(NS):
            acc = acc + cbuf[s, pl.ds(r * L, L)]
        counts1d[pl.ds(r * L, L)] = acc
        return 0

    lax.fori_loop(0, KROWS, sum_row, 0)

    def term_row(r, acc):
        c = counts1d[pl.ds(r * L, L)]
        inv = 1.0 / jnp.maximum(c, 1.0)
        for dd in range(CPT):
            s = bucket[pl.ds(dd * K + r * L, L)]
            acc = acc + s * s * inv
        return acc

    acc = lax.fori_loop(0, KROWS, term_row, zeros)

    res_v[...] = sq - acc
    pltpu.sync_copy(res_v, out.at[wid])


def kernel(embedding, superpixel):
    parts = _penalty_kernel(embedding, superpixel.astype(jnp.int32))
    return jnp.sum(parts)
